# Initial kernel scaffold; baseline (speedup 1.0000x reference)
import jax
import jax.numpy as jnp
from jax.experimental import pallas as pl


def kernel(queries, keys, values):
    raise NotImplementedError("write your pallas kernel here")


# speedup vs baseline: 3.1024x; 3.1024x over previous
import functools
import math

import jax
import jax.numpy as jnp
from jax import lax
from jax.experimental import pallas as pl
from jax.experimental.pallas import tpu as pltpu
from jax.experimental.pallas import tpu_sc as plsc

_FRACTION = 0.33
_MIN32 = -2147483648


def _f32_ord(x):
    i = lax.bitcast_convert_type(x, jnp.int32)
    mask = lax.shift_right_arithmetic(i, 31) & jnp.int32(0x7FFFFFFF)
    return i ^ mask


def _ord_f32(i):
    mask = lax.shift_right_arithmetic(i, 31) & jnp.int32(0x7FFFFFFF)
    return lax.bitcast_convert_type(i ^ mask, jnp.float32)


def _kth_largest_cols(ik, k):
    C = ik.shape[1]
    cnt0 = jnp.sum((ik >= 0).astype(jnp.int32), axis=0, keepdims=True)
    t = jnp.where(cnt0 >= k, jnp.zeros((1, C), jnp.int32),
                  jnp.full((1, C), _MIN32, jnp.int32))
    for bit in range(30, -1, -1):
        cand = t | jnp.int32(1 << bit)
        cnt = jnp.sum((ik >= cand).astype(jnp.int32), axis=0, keepdims=True)
        t = jnp.where(cnt >= k, cand, t)
    return t


def _kth_largest_all(ik, k):
    cnt0 = jnp.sum((ik >= 0).astype(jnp.int32))
    t = jnp.where(cnt0 >= k, jnp.int32(0), jnp.int32(_MIN32))
    for bit in range(30, -1, -1):
        cand = t | jnp.int32(1 << bit)
        cnt = jnp.sum((ik >= cand).astype(jnp.int32))
        t = jnp.where(cnt >= k, cand, t)
    return t


def _cumsum_rowmajor(x2d):
    R, C = x2d.shape
    xf = x2d.astype(jnp.float32)
    tri_c = (lax.broadcasted_iota(jnp.int32, (C, C), 0)
             <= lax.broadcasted_iota(jnp.int32, (C, C), 1)).astype(jnp.float32)
    within = lax.dot_general(xf, tri_c, (((1,), (0,)), ((), ())),
                             preferred_element_type=jnp.float32)
    rowsum = jnp.sum(xf, axis=1, keepdims=True)
    tri_r = (lax.broadcasted_iota(jnp.int32, (R, R), 1)
             < lax.broadcasted_iota(jnp.int32, (R, R), 0)).astype(jnp.float32)
    offs = jnp.sum(tri_r * rowsum.reshape(1, R), axis=1, keepdims=True)
    return (within + offs).astype(jnp.int32)


def _stats_body(keys_ref, values_ref, kred_ref, mv_ref, *, k):
    kb = keys_ref[0]
    vb = values_ref[0]
    n = kb.shape[0]
    mv_ref[0] = jnp.sum(vb, axis=0, keepdims=True) * (1.0 / n)
    ik = _f32_ord(kb)
    t = _kth_largest_cols(ik, k)
    gt = ik > t
    cnt_gt = jnp.sum(gt.astype(jnp.float32), axis=0, keepdims=True)
    sum_gt = jnp.sum(jnp.where(gt, kb, 0.0), axis=0, keepdims=True)
    tval = _ord_f32(t)
    topk_sum = sum_gt + tval * (jnp.float32(k) - cnt_gt)
    kred_ref[0] = topk_sum * (1.0 / k)


def _select_body(q_ref, kred_ref, g_ref, *, k, dump, batch_stride):
    b = pl.program_id(0)
    q = q_ref[0]
    kr = kred_ref[0]
    krb = jnp.broadcast_to(kr.astype(jnp.bfloat16), (8, kr.shape[1]))
    scores = lax.dot_general(
        q.astype(jnp.bfloat16), krb,
        (((1,), (1,)), ((), ())), preferred_element_type=jnp.float32)
    L = q.shape[0]
    s2 = scores[:, 0:1].reshape(L // 128, 128)
    ik = _f32_ord(s2)
    t = _kth_largest_all(ik, k)
    gt = ik > t
    cnt_gt = jnp.sum(gt.astype(jnp.int32))
    eq = ik == t
    eq_rank = _cumsum_rowmajor(eq)
    mask = gt | (eq & (eq_rank <= (k - cnt_gt)))
    pos = _cumsum_rowmajor(mask) - 1
    g = jnp.where(mask, pos, jnp.int32(dump)) + b * jnp.int32(batch_stride)
    g_ref[0] = g.reshape(1, L)


def _attn_body(q_ref, k_ref, v_ref, mv_ref, o_ref, acc_ref, m_ref, l_ref, *,
               scale, lq, nk, qb):
    ki = pl.program_id(2)

    @pl.when(ki == 0)
    def _init():
        m_ref[...] = jnp.full_like(m_ref, -jnp.inf)
        l_ref[...] = jnp.zeros_like(l_ref)
        acc_ref[...] = jnp.zeros_like(acc_ref)

    q = q_ref[0].astype(jnp.bfloat16)
    kk = k_ref[0].astype(jnp.bfloat16)
    s = lax.dot_general(q, kk, (((1,), (1,)), ((), ())),
                        preferred_element_type=jnp.float32) * scale
    m_prev = m_ref[:, 0:1]
    m_new = jnp.maximum(m_prev, jnp.max(s, axis=1, keepdims=True))
    alpha = jnp.exp(m_prev - m_new)
    p = jnp.exp(s - m_new)
    l_new = l_ref[:, 0:1] * alpha + jnp.sum(p, axis=1, keepdims=True)
    acc = acc_ref[...] * alpha + lax.dot_general(
        p.astype(jnp.bfloat16), v_ref[0].astype(jnp.bfloat16),
        (((1,), (0,)), ((), ())), preferred_element_type=jnp.float32)
    m_ref[...] = jnp.broadcast_to(m_new, m_ref.shape)
    l_ref[...] = jnp.broadcast_to(l_new, l_ref.shape)
    acc_ref[...] = acc

    @pl.when(ki == nk - 1)
    def _fin():
        out = acc_ref[...] / l_ref[:, 0:1]
        qi = pl.program_id(1)
        rows = qi * qb + lax.broadcasted_iota(jnp.int32, (qb, 1), 0)
        o_ref[0] = jnp.where(rows >= lq, mv_ref[0], out)


def _make_sc_kernels(n_rows, n_tab_rows, d, nw, ch):
    per_w = n_rows // nw
    n_ch = per_w // ch
    mesh = plsc.VectorSubcoreMesh(core_axis_name="c", subcore_axis_name="s")
    info = plsc.get_sparse_core_info()
    nc = info.num_cores
    scratch = [pltpu.VMEM((ch,), jnp.int32),
               pltpu.VMEM((ch, d), jnp.float32),
               pltpu.SemaphoreType.DMA]

    @functools.partial(
        pl.kernel, mesh=mesh,
        out_type=jax.ShapeDtypeStruct((n_tab_rows, d), jnp.float32),
        scratch_types=scratch)
    def scatter_rows(src_hbm, g_hbm, tab_hbm, idx_v, rows_v, sem):
        wid = lax.axis_index("s") * nc + lax.axis_index("c")
        base_w = wid * per_w

        def chunk(i, carry):
            base = base_w + i * ch
            pltpu.sync_copy(g_hbm.at[pl.ds(base, ch)], idx_v)
            pltpu.sync_copy(src_hbm.at[pl.ds(base, ch)], rows_v)
            pltpu.async_copy(rows_v, tab_hbm.at[idx_v], sem).wait()
            return carry

        lax.fori_loop(0, n_ch, chunk, 0)

    @functools.partial(
        pl.kernel, mesh=mesh,
        out_type=jax.ShapeDtypeStruct((n_rows, d), jnp.float32),
        scratch_types=scratch)
    def gather_rows(tab_hbm, g_hbm, out_hbm, idx_v, rows_v, sem):
        wid = lax.axis_index("s") * nc + lax.axis_index("c")
        base_w = wid * per_w

        def chunk(i, carry):
            base = base_w + i * ch
            pltpu.sync_copy(g_hbm.at[pl.ds(base, ch)], idx_v)
            pltpu.async_copy(tab_hbm.at[idx_v], rows_v, sem).wait()
            pltpu.sync_copy(rows_v, out_hbm.at[pl.ds(base, ch)])
            return carry

        lax.fori_loop(0, n_ch, chunk, 0)

    return scatter_rows, gather_rows


def kernel(queries, keys, values):
    B, L_Q, D = queries.shape
    L_K = keys.shape[1]
    l_q = int((1.0 - _FRACTION) * L_Q)
    QB = 256
    LQP = (l_q // QB + 1) * QB
    KB = 512
    BD = 256
    scale = 1.0 / math.sqrt(D)

    kred, mv = pl.pallas_call(
        functools.partial(_stats_body, k=l_q),
        grid=(B, D // BD),
        in_specs=[
            pl.BlockSpec((1, L_K, BD), lambda b, j: (b, 0, j)),
            pl.BlockSpec((1, L_K, BD), lambda b, j: (b, 0, j)),
        ],
        out_specs=[
            pl.BlockSpec((1, 1, BD), lambda b, j: (b, 0, j)),
            pl.BlockSpec((1, 1, BD), lambda b, j: (b, 0, j)),
        ],
        out_shape=[
            jax.ShapeDtypeStruct((B, 1, D), jnp.float32),
            jax.ShapeDtypeStruct((B, 1, D), jnp.float32),
        ],
    )(keys, values)

    g = pl.pallas_call(
        functools.partial(_select_body, k=l_q, dump=l_q, batch_stride=LQP),
        grid=(B,),
        in_specs=[
            pl.BlockSpec((1, L_Q, D), lambda b: (b, 0, 0)),
            pl.BlockSpec((1, 1, D), lambda b: (b, 0, 0)),
        ],
        out_specs=pl.BlockSpec((1, 1, L_Q), lambda b: (b, 0, 0)),
        out_shape=jax.ShapeDtypeStruct((B, 1, L_Q), jnp.int32),
    )(queries, kred)

    gflat = g.reshape(B * L_Q)
    n_rows = B * L_Q
    n_tab = B * LQP
    scatter_rows, gather_rows = _make_sc_kernels(n_rows, n_tab, D, 32, 64)

    q_sample = scatter_rows(queries.reshape(n_rows, D), gflat)

    attn = pl.pallas_call(
        functools.partial(_attn_body, scale=scale, lq=l_q, nk=L_K // KB,
                          qb=QB),
        grid=(B, LQP // QB, L_K // KB),
        in_specs=[
            pl.BlockSpec((1, QB, D), lambda b, qi, ki: (b, qi, 0)),
            pl.BlockSpec((1, KB, D), lambda b, qi, ki: (b, ki, 0)),
            pl.BlockSpec((1, KB, D), lambda b, qi, ki: (b, ki, 0)),
            pl.BlockSpec((1, 1, D), lambda b, qi, ki: (b, 0, 0)),
        ],
        out_specs=pl.BlockSpec((1, QB, D), lambda b, qi, ki: (b, qi, 0)),
        out_shape=jax.ShapeDtypeStruct((B, LQP, D), jnp.float32),
        scratch_shapes=[
            pltpu.VMEM((QB, D), jnp.float32),
            pltpu.VMEM((QB, 128), jnp.float32),
            pltpu.VMEM((QB, 128), jnp.float32),
        ],
        compiler_params=pltpu.CompilerParams(
            dimension_semantics=("parallel", "parallel", "arbitrary")),
    )(q_sample.reshape(B, LQP, D), keys, values, mv)

    out_flat = gather_rows(attn.reshape(n_tab, D), gflat)
    return out_flat.reshape(B, L_Q, D)


# speedup vs baseline: 4.4352x; 1.4296x over previous
import functools
import math

import jax
import jax.numpy as jnp
from jax import lax
from jax.experimental import pallas as pl
from jax.experimental.pallas import tpu as pltpu
from jax.experimental.pallas import tpu_sc as plsc

_FRACTION = 0.33
_MIN32 = -2147483648


def _f32_ord(x):
    i = lax.bitcast_convert_type(x, jnp.int32)
    mask = lax.shift_right_arithmetic(i, 31) & jnp.int32(0x7FFFFFFF)
    return i ^ mask


def _count_ge(ik, cand):
    col = jnp.sum((ik >= cand).astype(jnp.float32), axis=0, keepdims=True)
    return jnp.sum(col)


def _kth_largest_all(ik, k):
    kf = jnp.float32(k)
    t = jnp.where(_count_ge(ik, jnp.int32(0)) >= kf,
                  jnp.int32(0), jnp.int32(_MIN32))
    for bit in range(30, -1, -1):
        cand = t | jnp.int32(1 << bit)
        t = jnp.where(_count_ge(ik, cand) >= kf, cand, t)
    return t


def _cumsum_rowmajor(x2d):
    R, C = x2d.shape
    xf = x2d.astype(jnp.float32)
    tri_c = (lax.broadcasted_iota(jnp.int32, (C, C), 0)
             <= lax.broadcasted_iota(jnp.int32, (C, C), 1)).astype(jnp.float32)
    within = lax.dot_general(xf, tri_c, (((1,), (0,)), ((), ())),
                             preferred_element_type=jnp.float32)
    rowsum = jnp.sum(xf, axis=1, keepdims=True)
    tri_r = (lax.broadcasted_iota(jnp.int32, (R, R), 1)
             < lax.broadcasted_iota(jnp.int32, (R, R), 0)).astype(jnp.float32)
    offs = jnp.sum(tri_r * rowsum.reshape(1, R), axis=1, keepdims=True)
    return (within + offs).astype(jnp.int32)


def _stats_body(keys_ref, values_ref, kred_ref, mv_ref, kbf_ref, vbf_ref, *,
                k, n_iter):
    kb = keys_ref[0]
    vb = values_ref[0]
    n = kb.shape[0]
    mv_ref[0] = jnp.sum(vb, axis=0, keepdims=True) * (1.0 / n)
    kbf_ref[0] = kb.astype(jnp.bfloat16)
    vbf_ref[0] = vb.astype(jnp.bfloat16)
    lo = jnp.min(kb, axis=0, keepdims=True)
    hi = jnp.max(kb, axis=0, keepdims=True)
    kf = jnp.float32(k)
    for _ in range(n_iter):
        mid = 0.5 * (lo + hi)
        cnt = jnp.sum((kb >= mid).astype(jnp.float32), axis=0, keepdims=True)
        pred = cnt >= kf
        lo = jnp.where(pred, mid, lo)
        hi = jnp.where(pred, hi, mid)
    t = lo
    gt = kb > t
    cnt_gt = jnp.sum(gt.astype(jnp.float32), axis=0, keepdims=True)
    sum_gt = jnp.sum(jnp.where(gt, kb, 0.0), axis=0, keepdims=True)
    topk_sum = sum_gt + t * (kf - cnt_gt)
    kred_ref[0] = topk_sum * (1.0 / k)


def _select_body(q_ref, kred_ref, g_ref, *, k, dump, batch_stride):
    b = pl.program_id(0)
    q = q_ref[0]
    kr = kred_ref[0]
    krb = jnp.broadcast_to(kr.astype(jnp.bfloat16), (8, kr.shape[1]))
    scores = lax.dot_general(
        q.astype(jnp.bfloat16), krb,
        (((1,), (1,)), ((), ())), preferred_element_type=jnp.float32)
    L = q.shape[0]
    s2 = scores[:, 0:1].reshape(L // 128, 128)
    ik = _f32_ord(s2)
    t = _kth_largest_all(ik, k)
    gt = ik > t
    cnt_gt = jnp.sum(gt.astype(jnp.int32))
    eq = ik == t
    eq_rank = _cumsum_rowmajor(eq)
    mask = gt | (eq & (eq_rank <= (k - cnt_gt)))
    pos = _cumsum_rowmajor(mask) - 1
    g = jnp.where(mask, pos, jnp.int32(dump)) + b * jnp.int32(batch_stride)
    g_ref[0] = g.reshape(1, L)


def _attn_body(q_ref, k_ref, v_ref, mv_ref, o_ref, acc_ref, m_ref, l_ref, *,
               scale, lq, nk, qb):
    ki = pl.program_id(2)

    @pl.when(ki == 0)
    def _init():
        m_ref[...] = jnp.full_like(m_ref, -jnp.inf)
        l_ref[...] = jnp.zeros_like(l_ref)
        acc_ref[...] = jnp.zeros_like(acc_ref)

    q = q_ref[0].astype(jnp.bfloat16)
    kk = k_ref[0]
    s = lax.dot_general(q, kk, (((1,), (1,)), ((), ())),
                        preferred_element_type=jnp.float32) * scale
    m_prev = m_ref[:, 0:1]
    m_new = jnp.maximum(m_prev, jnp.max(s, axis=1, keepdims=True))
    alpha = jnp.exp(m_prev - m_new)
    p = jnp.exp(s - m_new)
    l_new = l_ref[:, 0:1] * alpha + jnp.sum(p, axis=1, keepdims=True)
    acc = acc_ref[...] * alpha + lax.dot_general(
        p.astype(jnp.bfloat16), v_ref[0],
        (((1,), (0,)), ((), ())), preferred_element_type=jnp.float32)
    m_ref[...] = jnp.broadcast_to(m_new, m_ref.shape)
    l_ref[...] = jnp.broadcast_to(l_new, l_ref.shape)
    acc_ref[...] = acc

    @pl.when(ki == nk - 1)
    def _fin():
        out = acc_ref[...] / l_ref[:, 0:1]
        qi = pl.program_id(1)
        rows = qi * qb + lax.broadcasted_iota(jnp.int32, (qb, 1), 0)
        o_ref[0] = jnp.where(rows >= lq, mv_ref[0], out)


def _make_sc_kernels(n_rows, n_tab_rows, d, nw, ch):
    per_w = n_rows // nw
    n_ch = per_w // ch
    mesh = plsc.VectorSubcoreMesh(core_axis_name="c", subcore_axis_name="s")
    info = plsc.get_sparse_core_info()
    nc = info.num_cores
    scratch = [pltpu.VMEM((ch,), jnp.int32),
               pltpu.VMEM((ch, d), jnp.float32),
               pltpu.SemaphoreType.DMA]

    @functools.partial(
        pl.kernel, mesh=mesh,
        out_type=jax.ShapeDtypeStruct((n_tab_rows, d), jnp.float32),
        scratch_types=scratch)
    def scatter_rows(src_hbm, g_hbm, tab_hbm, idx_v, rows_v, sem):
        wid = lax.axis_index("s") * nc + lax.axis_index("c")
        base_w = wid * per_w

        def chunk(i, carry):
            base = base_w + i * ch
            pltpu.sync_copy(g_hbm.at[pl.ds(base, ch)], idx_v)
            pltpu.sync_copy(src_hbm.at[pl.ds(base, ch)], rows_v)
            pltpu.async_copy(rows_v, tab_hbm.at[idx_v], sem).wait()
            return carry

        lax.fori_loop(0, n_ch, chunk, 0)

    @functools.partial(
        pl.kernel, mesh=mesh,
        out_type=jax.ShapeDtypeStruct((n_rows, d), jnp.float32),
        scratch_types=scratch)
    def gather_rows(tab_hbm, g_hbm, out_hbm, idx_v, rows_v, sem):
        wid = lax.axis_index("s") * nc + lax.axis_index("c")
        base_w = wid * per_w

        def chunk(i, carry):
            base = base_w + i * ch
            pltpu.sync_copy(g_hbm.at[pl.ds(base, ch)], idx_v)
            pltpu.async_copy(tab_hbm.at[idx_v], rows_v, sem).wait()
            pltpu.sync_copy(rows_v, out_hbm.at[pl.ds(base, ch)])
            return carry

        lax.fori_loop(0, n_ch, chunk, 0)

    return scatter_rows, gather_rows


def kernel(queries, keys, values):
    B, L_Q, D = queries.shape
    L_K = keys.shape[1]
    l_q = int((1.0 - _FRACTION) * L_Q)
    QB = 256
    LQP = (l_q // QB + 1) * QB
    KB = 1024
    BD = 256
    scale = 1.0 / math.sqrt(D)

    kred, mv, keys_bf, values_bf = pl.pallas_call(
        functools.partial(_stats_body, k=l_q, n_iter=16),
        grid=(B, D // BD),
        in_specs=[
            pl.BlockSpec((1, L_K, BD), lambda b, j: (b, 0, j)),
            pl.BlockSpec((1, L_K, BD), lambda b, j: (b, 0, j)),
        ],
        out_specs=[
            pl.BlockSpec((1, 1, BD), lambda b, j: (b, 0, j)),
            pl.BlockSpec((1, 1, BD), lambda b, j: (b, 0, j)),
            pl.BlockSpec((1, L_K, BD), lambda b, j: (b, 0, j)),
            pl.BlockSpec((1, L_K, BD), lambda b, j: (b, 0, j)),
        ],
        out_shape=[
            jax.ShapeDtypeStruct((B, 1, D), jnp.float32),
            jax.ShapeDtypeStruct((B, 1, D), jnp.float32),
            jax.ShapeDtypeStruct((B, L_K, D), jnp.bfloat16),
            jax.ShapeDtypeStruct((B, L_K, D), jnp.bfloat16),
        ],
    )(keys, values)

    g = pl.pallas_call(
        functools.partial(_select_body, k=l_q, dump=l_q, batch_stride=LQP),
        grid=(B,),
        in_specs=[
            pl.BlockSpec((1, L_Q, D), lambda b: (b, 0, 0)),
            pl.BlockSpec((1, 1, D), lambda b: (b, 0, 0)),
        ],
        out_specs=pl.BlockSpec((1, 1, L_Q), lambda b: (b, 0, 0)),
        out_shape=jax.ShapeDtypeStruct((B, 1, L_Q), jnp.int32),
    )(queries, kred)

    gflat = g.reshape(B * L_Q)
    n_rows = B * L_Q
    n_tab = B * LQP
    scatter_rows, gather_rows = _make_sc_kernels(n_rows, n_tab, D, 32, 64)

    q_sample = scatter_rows(queries.reshape(n_rows, D), gflat)

    attn = pl.pallas_call(
        functools.partial(_attn_body, scale=scale, lq=l_q, nk=L_K // KB,
                          qb=QB),
        grid=(B, LQP // QB, L_K // KB),
        in_specs=[
            pl.BlockSpec((1, QB, D), lambda b, qi, ki: (b, qi, 0)),
            pl.BlockSpec((1, KB, D), lambda b, qi, ki: (b, ki, 0)),
            pl.BlockSpec((1, KB, D), lambda b, qi, ki: (b, ki, 0)),
            pl.BlockSpec((1, 1, D), lambda b, qi, ki: (b, 0, 0)),
        ],
        out_specs=pl.BlockSpec((1, QB, D), lambda b, qi, ki: (b, qi, 0)),
        out_shape=jax.ShapeDtypeStruct((B, LQP, D), jnp.float32),
        scratch_shapes=[
            pltpu.VMEM((QB, D), jnp.float32),
            pltpu.VMEM((QB, 128), jnp.float32),
            pltpu.VMEM((QB, 128), jnp.float32),
        ],
        compiler_params=pltpu.CompilerParams(
            dimension_semantics=("parallel", "parallel", "arbitrary")),
    )(q_sample.reshape(B, LQP, D), keys_bf, values_bf, mv)

    out_flat = gather_rows(attn.reshape(n_tab, D), gflat)
    return out_flat.reshape(B, L_Q, D)


# speedup vs baseline: 4.4762x; 1.0093x over previous
import functools
import math

import jax
import jax.numpy as jnp
from jax import lax
from jax.experimental import pallas as pl
from jax.experimental.pallas import tpu as pltpu
from jax.experimental.pallas import tpu_sc as plsc

_FRACTION = 0.33
_MIN32 = -2147483648


def _f32_ord(x):
    i = lax.bitcast_convert_type(x, jnp.int32)
    mask = lax.shift_right_arithmetic(i, 31) & jnp.int32(0x7FFFFFFF)
    return i ^ mask


def _count_ge(ik, cand):
    col = jnp.sum((ik >= cand).astype(jnp.float32), axis=0, keepdims=True)
    return jnp.sum(col)


def _kth_largest_all(ik, k):
    kf = jnp.float32(k)
    t = jnp.where(_count_ge(ik, jnp.int32(0)) >= kf,
                  jnp.int32(0), jnp.int32(_MIN32))
    for bit in range(30, -1, -1):
        cand = t | jnp.int32(1 << bit)
        t = jnp.where(_count_ge(ik, cand) >= kf, cand, t)
    return t


def _cumsum_rowmajor(x2d):
    R, C = x2d.shape
    xf = x2d.astype(jnp.float32)
    tri_c = (lax.broadcasted_iota(jnp.int32, (C, C), 0)
             <= lax.broadcasted_iota(jnp.int32, (C, C), 1)).astype(jnp.float32)
    within = lax.dot_general(xf, tri_c, (((1,), (0,)), ((), ())),
                             preferred_element_type=jnp.float32)
    rowsum = jnp.sum(xf, axis=1, keepdims=True)
    tri_r = (lax.broadcasted_iota(jnp.int32, (R, R), 1)
             < lax.broadcasted_iota(jnp.int32, (R, R), 0)).astype(jnp.float32)
    offs = jnp.sum(tri_r * rowsum.reshape(1, R), axis=1, keepdims=True)
    return (within + offs).astype(jnp.int32)


def _stats_body(keys_ref, values_ref, kred_ref, mv_ref, kbf_ref, vbf_ref, *,
                k, n_iter):
    kb = keys_ref[0]
    vb = values_ref[0]
    n = kb.shape[0]
    mv_ref[0] = jnp.sum(vb, axis=0, keepdims=True) * (1.0 / n)
    kbf_ref[0] = kb.astype(jnp.bfloat16)
    vbf_ref[0] = vb.astype(jnp.bfloat16)
    lo = jnp.min(kb, axis=0, keepdims=True)
    hi = jnp.max(kb, axis=0, keepdims=True)
    kf = jnp.float32(k)
    for _ in range(n_iter):
        mid = 0.5 * (lo + hi)
        cnt = jnp.sum((kb >= mid).astype(jnp.float32), axis=0, keepdims=True)
        pred = cnt >= kf
        lo = jnp.where(pred, mid, lo)
        hi = jnp.where(pred, hi, mid)
    t = lo
    gt = kb > t
    cnt_gt = jnp.sum(gt.astype(jnp.float32), axis=0, keepdims=True)
    sum_gt = jnp.sum(jnp.where(gt, kb, 0.0), axis=0, keepdims=True)
    topk_sum = sum_gt + t * (kf - cnt_gt)
    kred_ref[0] = topk_sum * (1.0 / k)


def _select_body(q_ref, kred_ref, g_ref, *, k, dump, batch_stride):
    b = pl.program_id(0)
    q = q_ref[0]
    kr = kred_ref[0]
    krb = jnp.broadcast_to(kr.astype(jnp.bfloat16), (8, kr.shape[1]))
    scores = lax.dot_general(
        q.astype(jnp.bfloat16), krb,
        (((1,), (1,)), ((), ())), preferred_element_type=jnp.float32)
    L = q.shape[0]
    s2 = scores[:, 0:1].reshape(L // 128, 128)
    ik = _f32_ord(s2)
    t = _kth_largest_all(ik, k)
    gt = ik > t
    cnt_gt = jnp.sum(gt.astype(jnp.int32))
    eq = ik == t
    eq_rank = _cumsum_rowmajor(eq)
    mask = gt | (eq & (eq_rank <= (k - cnt_gt)))
    pos = _cumsum_rowmajor(mask) - 1
    g = jnp.where(mask, pos, jnp.int32(dump)) + b * jnp.int32(batch_stride)
    g_ref[0] = g.reshape(1, L)


def _attn_body(q_ref, k_ref, v_ref, mv_ref, o_ref, acc_ref, m_ref, l_ref, *,
               scale, lq, nk, qb):
    ki = pl.program_id(2)

    @pl.when(ki == 0)
    def _init():
        m_ref[...] = jnp.full_like(m_ref, -jnp.inf)
        l_ref[...] = jnp.zeros_like(l_ref)
        acc_ref[...] = jnp.zeros_like(acc_ref)

    q = q_ref[0].astype(jnp.bfloat16)
    kk = k_ref[0]
    s = lax.dot_general(q, kk, (((1,), (1,)), ((), ())),
                        preferred_element_type=jnp.float32) * scale
    m_prev = m_ref[:, 0:1]
    m_new = jnp.maximum(m_prev, jnp.max(s, axis=1, keepdims=True))
    alpha = jnp.exp(m_prev - m_new)
    p = jnp.exp(s - m_new)
    l_new = l_ref[:, 0:1] * alpha + jnp.sum(p, axis=1, keepdims=True)
    acc = acc_ref[...] * alpha + lax.dot_general(
        p.astype(jnp.bfloat16), v_ref[0],
        (((1,), (0,)), ((), ())), preferred_element_type=jnp.float32)
    m_ref[...] = jnp.broadcast_to(m_new, m_ref.shape)
    l_ref[...] = jnp.broadcast_to(l_new, l_ref.shape)
    acc_ref[...] = acc

    @pl.when(ki == nk - 1)
    def _fin():
        out = acc_ref[...] / l_ref[:, 0:1]
        qi = pl.program_id(1)
        rows = qi * qb + lax.broadcasted_iota(jnp.int32, (qb, 1), 0)
        o_ref[0] = jnp.where(rows >= lq, mv_ref[0], out)


def _make_sc_kernels(n_rows, n_tab_rows, d, nw, ch):
    per_w = n_rows // nw
    n_ch = per_w // ch
    mesh = plsc.VectorSubcoreMesh(core_axis_name="c", subcore_axis_name="s")
    info = plsc.get_sparse_core_info()
    nc = info.num_cores
    scratch = [pltpu.VMEM((n_ch, ch), jnp.int32),
               pltpu.VMEM((ch, d), jnp.float32),
               pltpu.VMEM((ch, d), jnp.float32),
               pltpu.SemaphoreType.DMA,
               pltpu.SemaphoreType.DMA]

    @functools.partial(
        pl.kernel, mesh=mesh,
        out_type=jax.ShapeDtypeStruct((n_tab_rows, d), jnp.float32),
        scratch_types=scratch)
    def scatter_rows(src_hbm, g2_hbm, tab_hbm, idx_v, buf0, buf1, sem_r,
                     sem_w):
        wid = lax.axis_index("s") * nc + lax.axis_index("c")
        base_w = wid * per_w
        pltpu.sync_copy(g2_hbm.at[pl.ds(wid * n_ch, n_ch)], idx_v)
        bufs = (buf0, buf1)

        def rd(i):
            return pltpu.make_async_copy(
                src_hbm.at[pl.ds(base_w + i * ch, ch)], bufs[i % 2], sem_r)

        def wr(i):
            return pltpu.make_async_copy(
                bufs[i % 2], tab_hbm.at[idx_v.at[i]], sem_w)

        rd(0).start()
        rd(1).start()
        for i in range(n_ch):
            rd(i).wait()
            wr(i).start()
            wr(i).wait()
            if i + 2 < n_ch:
                rd(i + 2).start()

    @functools.partial(
        pl.kernel, mesh=mesh,
        out_type=jax.ShapeDtypeStruct((n_rows, d), jnp.float32),
        scratch_types=scratch)
    def gather_rows(tab_hbm, g2_hbm, out_hbm, idx_v, buf0, buf1, sem_r,
                    sem_w):
        wid = lax.axis_index("s") * nc + lax.axis_index("c")
        base_w = wid * per_w
        pltpu.sync_copy(g2_hbm.at[pl.ds(wid * n_ch, n_ch)], idx_v)
        bufs = (buf0, buf1)

        def rd(i):
            return pltpu.make_async_copy(
                tab_hbm.at[idx_v.at[i]], bufs[i % 2], sem_r)

        def wr(i):
            return pltpu.make_async_copy(
                bufs[i % 2], out_hbm.at[pl.ds(base_w + i * ch, ch)], sem_w)

        rd(0).start()
        rd(1).start()
        for i in range(n_ch):
            rd(i).wait()
            wr(i).start()
            wr(i).wait()
            if i + 2 < n_ch:
                rd(i + 2).start()

    return scatter_rows, gather_rows


def kernel(queries, keys, values):
    B, L_Q, D = queries.shape
    L_K = keys.shape[1]
    l_q = int((1.0 - _FRACTION) * L_Q)
    QB = 256
    LQP = (l_q // QB + 1) * QB
    KB = 1024
    BD = 256
    scale = 1.0 / math.sqrt(D)

    kred, mv, keys_bf, values_bf = pl.pallas_call(
        functools.partial(_stats_body, k=l_q, n_iter=16),
        grid=(B, D // BD),
        in_specs=[
            pl.BlockSpec((1, L_K, BD), lambda b, j: (b, 0, j)),
            pl.BlockSpec((1, L_K, BD), lambda b, j: (b, 0, j)),
        ],
        out_specs=[
            pl.BlockSpec((1, 1, BD), lambda b, j: (b, 0, j)),
            pl.BlockSpec((1, 1, BD), lambda b, j: (b, 0, j)),
            pl.BlockSpec((1, L_K, BD), lambda b, j: (b, 0, j)),
            pl.BlockSpec((1, L_K, BD), lambda b, j: (b, 0, j)),
        ],
        out_shape=[
            jax.ShapeDtypeStruct((B, 1, D), jnp.float32),
            jax.ShapeDtypeStruct((B, 1, D), jnp.float32),
            jax.ShapeDtypeStruct((B, L_K, D), jnp.bfloat16),
            jax.ShapeDtypeStruct((B, L_K, D), jnp.bfloat16),
        ],
    )(keys, values)

    g = pl.pallas_call(
        functools.partial(_select_body, k=l_q, dump=l_q, batch_stride=LQP),
        grid=(B,),
        in_specs=[
            pl.BlockSpec((1, L_Q, D), lambda b: (b, 0, 0)),
            pl.BlockSpec((1, 1, D), lambda b: (b, 0, 0)),
        ],
        out_specs=pl.BlockSpec((1, 1, L_Q), lambda b: (b, 0, 0)),
        out_shape=jax.ShapeDtypeStruct((B, 1, L_Q), jnp.int32),
    )(queries, kred)

    n_rows = B * L_Q
    n_tab = B * LQP
    CH = 32
    g2 = g.reshape(n_rows // CH, CH)
    scatter_rows, gather_rows = _make_sc_kernels(n_rows, n_tab, D, 32, CH)

    q_sample = scatter_rows(queries.reshape(n_rows, D), g2)

    attn = pl.pallas_call(
        functools.partial(_attn_body, scale=scale, lq=l_q, nk=L_K // KB,
                          qb=QB),
        grid=(B, LQP // QB, L_K // KB),
        in_specs=[
            pl.BlockSpec((1, QB, D), lambda b, qi, ki: (b, qi, 0)),
            pl.BlockSpec((1, KB, D), lambda b, qi, ki: (b, ki, 0)),
            pl.BlockSpec((1, KB, D), lambda b, qi, ki: (b, ki, 0)),
            pl.BlockSpec((1, 1, D), lambda b, qi, ki: (b, 0, 0)),
        ],
        out_specs=pl.BlockSpec((1, QB, D), lambda b, qi, ki: (b, qi, 0)),
        out_shape=jax.ShapeDtypeStruct((B, LQP, D), jnp.float32),
        scratch_shapes=[
            pltpu.VMEM((QB, D), jnp.float32),
            pltpu.VMEM((QB, 128), jnp.float32),
            pltpu.VMEM((QB, 128), jnp.float32),
        ],
        compiler_params=pltpu.CompilerParams(
            dimension_semantics=("parallel", "parallel", "arbitrary")),
    )(q_sample.reshape(B, LQP, D), keys_bf, values_bf, mv)

    out_flat = gather_rows(attn.reshape(n_tab, D), g2)
    return out_flat.reshape(B, L_Q, D)
